# baseline (speedup 1.0000x reference)
import functools

import jax
import jax.numpy as jnp
from jax import lax
from jax.experimental import pallas as pl
from jax.experimental.pallas import tpu as pltpu
from jax.experimental.pallas import tpu_sc as plsc

NC, NS = 2, 16
NW = NC * NS
EB = 128
NBUF = 2
NIB = 4


def _pre_body(x_ref, labm_ref, wfct_ref, wfc2t_ref, mu_ref, sg_ref,
              vfr_ref, wu_ref):
    x = x_ref[...]
    z = jnp.dot(x, wfct_ref[...], preferred_element_type=jnp.float32)
    u = jnp.dot(x, wfc2t_ref[...], preferred_element_type=jnp.float32)
    t = (z - mu_ref[...]) * sg_ref[...]
    wu_ref[...] = jnp.exp(-0.5 * t * t) * u
    vfr_ref[...] = x * labm_ref[...]


def _pre(x, labm, wfct, wfc2t, mu_row, sg_row, block):
    n, d = x.shape
    grid = (n + block - 1) // block
    return pl.pallas_call(
        _pre_body,
        grid=(grid,),
        in_specs=[
            pl.BlockSpec((block, d), lambda i: (i, 0)),
            pl.BlockSpec((block, d), lambda i: (i, 0)),
            pl.BlockSpec((d, d), lambda i: (0, 0)),
            pl.BlockSpec((d, d), lambda i: (0, 0)),
            pl.BlockSpec((1, d), lambda i: (0, 0)),
            pl.BlockSpec((1, d), lambda i: (0, 0)),
        ],
        out_specs=[
            pl.BlockSpec((block, d), lambda i: (i, 0)),
            pl.BlockSpec((block, d), lambda i: (i, 0)),
        ],
        out_shape=[
            jax.ShapeDtypeStruct((n, d), jnp.float32),
            jax.ShapeDtypeStruct((n, d), jnp.float32),
        ],
    )(x, labm, wfct, wfc2t, mu_row, sg_row)


def _make_seg(n_pad, d, e_pad, nchunks):
    mesh = plsc.VectorSubcoreMesh(core_axis_name="c", subcore_axis_name="s",
                                  num_cores=NC, num_subcores=NS)
    rpw = n_pad // NS

    per_worker = e_pad // NW

    @functools.partial(
        pl.kernel,
        out_type=jax.ShapeDtypeStruct((3 * NC, n_pad, d), jnp.float32),
        mesh=mesh,
        scratch_types=[
            [pltpu.VMEM((EB,), jnp.int32)] * NIB,
            [pltpu.VMEM((EB,), jnp.int32)] * NIB,
            [pltpu.VMEM((EB, d), jnp.float32)] * NBUF,
            pltpu.VMEM_SHARED((n_pad, d), jnp.float32),
            [pltpu.SemaphoreType.DMA] * NIB,
            [pltpu.SemaphoreType.DMA] * NIB,
            [pltpu.SemaphoreType.DMA] * NBUF,
        ],
    )
    def seg(xt, vfrt, wut, srcp, dstp, zrows, out, ibs, ibd, rows, acc,
            isem_s, isem_d, gsem):
        c = lax.axis_index("c")
        s = lax.axis_index("s")
        r0 = s * rpw
        wid = c * NS + s
        ebase = wid * per_worker

        def idx_start(m, q):
            off = pl.multiple_of(ebase + m * EB, 8)
            pltpu.async_copy(srcp.at[pl.ds(off, EB)], ibs[q], isem_s[q])
            pltpu.async_copy(dstp.at[pl.ds(off, EB)], ibd[q], isem_d[q])

        def idx_wait_src(q):
            pltpu.make_async_copy(srcp.at[pl.ds(0, EB)], ibs[q],
                                  isem_s[q]).wait()

        def idx_wait_dst(q):
            pltpu.make_async_copy(dstp.at[pl.ds(0, EB)], ibd[q],
                                  isem_d[q]).wait()

        for ch, tab in enumerate((xt, vfrt, wut)):
            pltpu.sync_copy(zrows, acc.at[pl.ds(r0, rpw)])
            plsc.subcore_barrier()

            def body(i, carry, tab=tab):
                idx_start(i, 0)
                idx_wait_src(0)
                idx_wait_dst(0)
                pltpu.async_copy(tab.at[ibs[0]], rows[0], gsem[0]).wait()
                pltpu.sync_copy(rows[0], acc.at[ibd[0]], add=True)
                return carry

            lax.fori_loop(0, nchunks, body, 0)
            plsc.subcore_barrier()
            for cc in range(NC):
                @pl.when(c == cc)
                def _(ch=ch, cc=cc):
                    pltpu.sync_copy(acc.at[pl.ds(r0, rpw)],
                                    out.at[ch * NC + cc, pl.ds(r0, rpw)])
            plsc.subcore_barrier()

    return seg


def _post_body(x_ref, p_ref, wtfrt_ref, btfr_ref, wfrt_ref, bfr_ref,
               wtbet_ref, btbe_ref, wbet_ref, bbe_ref, wbal_ref, bbal_ref,
               wselft_ref, bself_ref, wfc3t_ref, out_ref):
    x = x_ref[...]
    tot = p_ref[0] + p_ref[1]
    fr = p_ref[2] + p_ref[3]
    kern = p_ref[4] + p_ref[5]
    be = tot - fr
    t_fr = jnp.dot(x, wtfrt_ref[...], preferred_element_type=jnp.float32) + btfr_ref[...]
    out_fr = jnp.maximum(
        jnp.dot(fr * t_fr, wfrt_ref[...], preferred_element_type=jnp.float32)
        + bfr_ref[...], 0.0)
    t_be = jnp.dot(x, wtbet_ref[...], preferred_element_type=jnp.float32) + btbe_ref[...]
    out_be = jnp.maximum(
        jnp.dot(be * t_be, wbet_ref[...], preferred_element_type=jnp.float32)
        + bbe_ref[...], 0.0)
    ball = (jnp.sum(x * wbal_ref[...], axis=1, keepdims=True)
            + bbal_ref[...][:, :1])
    bal = jax.nn.sigmoid(jnp.maximum(ball, 0.0))
    h = jnp.dot(x, wselft_ref[...], preferred_element_type=jnp.float32) + bself_ref[...]
    h = h + bal * out_be + (1.0 - bal) * out_fr
    h = h + jnp.dot(kern, wfc3t_ref[...], preferred_element_type=jnp.float32)
    out_ref[...] = jnp.where(h > 0, h, 0.01 * h)


def _post(x, p6, wtfrt, btfr, wfrt, bfr, wtbet, btbe, wbet, bbe, wbal, bbal,
          wselft, bself, wfc3t, block):
    n, d = x.shape
    grid = (n + block - 1) // block
    row = lambda i: (0, 0)
    return pl.pallas_call(
        _post_body,
        grid=(grid,),
        in_specs=[
            pl.BlockSpec((block, d), lambda i: (i, 0)),
            pl.BlockSpec((6, block, d), lambda i: (0, i, 0)),
            pl.BlockSpec((d, d), row), pl.BlockSpec((1, d), row),
            pl.BlockSpec((d, d), row), pl.BlockSpec((1, d), row),
            pl.BlockSpec((d, d), row), pl.BlockSpec((1, d), row),
            pl.BlockSpec((d, d), row), pl.BlockSpec((1, d), row),
            pl.BlockSpec((1, d), row), pl.BlockSpec((1, d), row),
            pl.BlockSpec((d, d), row), pl.BlockSpec((1, d), row),
            pl.BlockSpec((d, d), row),
        ],
        out_specs=pl.BlockSpec((block, d), lambda i: (i, 0)),
        out_shape=jax.ShapeDtypeStruct((n, d), jnp.float32),
    )(x, p6, wtfrt, btfr, wfrt, bfr, wtbet, btbe, wbet, bbe, wbal, bbal,
      wselft, bself, wfc3t)


def kernel(x, edge_index, labels, W_fc, W_fc2, W_fc3, mu, inv_sigma,
           W_fr, b_fr, Wt_fr, bt_fr, W_be, b_be, Wt_be, bt_be,
           W_bal, b_bal, W_self, b_self):
    n, d = x.shape
    e = edge_index.shape[1]

    src = edge_index[0].astype(jnp.int32)
    dst = edge_index[1].astype(jnp.int32)
    chunk = NW * EB * NIB
    e_pad = ((e + chunk - 1) // chunk) * chunk
    pad = e_pad - e
    if pad:
        src = jnp.concatenate([src, jnp.zeros((pad,), jnp.int32)])
        dst = jnp.concatenate([dst, jnp.full((pad,), n, jnp.int32)])
        n_min = n + 1
    else:
        n_min = n
    n_pad = ((n_min + NS * 8 - 1) // (NS * 8)) * (NS * 8)
    nchunks = e_pad // (NW * EB)

    labm = jnp.broadcast_to((labels == 1).astype(jnp.float32)[:, None], (n, d))
    mu_row = mu.reshape(1, d)
    sg_row = inv_sigma.reshape(1, d)

    vfr, wu = _pre(x, labm, W_fc.T, W_fc2.T, mu_row, sg_row, block=1000)

    zrows = jnp.zeros((n_pad // NS, d), jnp.float32)
    seg = _make_seg(n_pad, d, e_pad, nchunks)
    p = seg(x, vfr, wu, src, dst, zrows)
    p6 = p[:, :n, :] if n_pad != n else p

    rst = _post(
        x, p6,
        Wt_fr.T, bt_fr.reshape(1, d), W_fr.T, b_fr.reshape(1, d),
        Wt_be.T, bt_be.reshape(1, d), W_be.T, b_be.reshape(1, d),
        W_bal.reshape(1, d), jnp.broadcast_to(b_bal.reshape(1, 1), (1, d)),
        W_self.T, b_self.reshape(1, d), W_fc3.T, block=1000)
    return rst


# speedup vs baseline: 1.9801x; 1.9801x over previous
import functools

import jax
import jax.numpy as jnp
from jax import lax
from jax.experimental import pallas as pl
from jax.experimental.pallas import tpu as pltpu
from jax.experimental.pallas import tpu_sc as plsc

NC, NS = 2, 16
NW = NC * NS
EB = 112
KU = 3


def _pre_body(x_ref, labm_ref, wfct_ref, wfc2t_ref, mu_ref, sg_ref,
              vfr_ref, wu_ref):
    x = x_ref[...]
    z = jnp.dot(x, wfct_ref[...], preferred_element_type=jnp.float32)
    u = jnp.dot(x, wfc2t_ref[...], preferred_element_type=jnp.float32)
    t = (z - mu_ref[...]) * sg_ref[...]
    wu_ref[...] = jnp.exp(-0.5 * t * t) * u
    vfr_ref[...] = x * labm_ref[...]


def _pre(x, labm, wfct, wfc2t, mu_row, sg_row, block):
    n, d = x.shape
    grid = (n + block - 1) // block
    return pl.pallas_call(
        _pre_body,
        grid=(grid,),
        in_specs=[
            pl.BlockSpec((block, d), lambda i: (i, 0)),
            pl.BlockSpec((block, d), lambda i: (i, 0)),
            pl.BlockSpec((d, d), lambda i: (0, 0)),
            pl.BlockSpec((d, d), lambda i: (0, 0)),
            pl.BlockSpec((1, d), lambda i: (0, 0)),
            pl.BlockSpec((1, d), lambda i: (0, 0)),
        ],
        out_specs=[
            pl.BlockSpec((block, d), lambda i: (i, 0)),
            pl.BlockSpec((block, d), lambda i: (i, 0)),
        ],
        out_shape=[
            jax.ShapeDtypeStruct((n, d), jnp.float32),
            jax.ShapeDtypeStruct((n, d), jnp.float32),
        ],
    )(x, labm, wfct, wfc2t, mu_row, sg_row)


def _make_seg(n_pad, d, e_pad, nchunks):
    mesh = plsc.VectorSubcoreMesh(core_axis_name="c", subcore_axis_name="s",
                                  num_cores=NC, num_subcores=NS)
    rpw = n_pad // NS

    @functools.partial(
        pl.kernel,
        out_type=jax.ShapeDtypeStruct((3 * NC, n_pad, d), jnp.float32),
        mesh=mesh,
        scratch_types=[
            [pltpu.VMEM((2, EB), jnp.int32)] * KU,
            [pltpu.VMEM((EB, d), jnp.float32)] * KU,
            pltpu.VMEM_SHARED((n_pad, d), jnp.float32),
            [pltpu.SemaphoreType.DMA] * KU,
            [pltpu.SemaphoreType.DMA] * KU,
        ],
    )
    def seg(xt, vfrt, wut, idxp, zrows, out, ib, rows, acc, isem, gsem):
        c = lax.axis_index("c")
        s = lax.axis_index("s")
        r0 = s * rpw
        wid = c * NS + s
        cbase = wid * nchunks

        for ch, tab in enumerate((xt, vfrt, wut)):
            pltpu.sync_copy(zrows, acc.at[pl.ds(r0, rpw)])
            plsc.subcore_barrier()

            def body(k, carry, tab=tab):
                c0 = cbase + k * KU
                icpy = [pltpu.async_copy(idxp.at[c0 + j], ib[j], isem[j])
                        for j in range(KU)]
                gcpy = []
                for j in range(KU):
                    icpy[j].wait()
                    gcpy.append(pltpu.async_copy(tab.at[ib[j].at[0]], rows[j],
                                                 gsem[j]))
                for j in range(KU):
                    gcpy[j].wait()
                    pltpu.sync_copy(rows[j], acc.at[ib[j].at[1]], add=True)
                return carry

            lax.fori_loop(0, nchunks // KU, body, 0)
            plsc.subcore_barrier()
            for cc in range(NC):
                @pl.when(c == cc)
                def _(ch=ch, cc=cc):
                    pltpu.sync_copy(acc.at[pl.ds(r0, rpw)],
                                    out.at[ch * NC + cc, pl.ds(r0, rpw)])
            plsc.subcore_barrier()

    return seg


def _post_body(x_ref, p_ref, wtfrt_ref, btfr_ref, wfrt_ref, bfr_ref,
               wtbet_ref, btbe_ref, wbet_ref, bbe_ref, wbal_ref, bbal_ref,
               wselft_ref, bself_ref, wfc3t_ref, out_ref):
    x = x_ref[...]
    tot = p_ref[0] + p_ref[1]
    fr = p_ref[2] + p_ref[3]
    kern = p_ref[4] + p_ref[5]
    be = tot - fr
    t_fr = jnp.dot(x, wtfrt_ref[...], preferred_element_type=jnp.float32) + btfr_ref[...]
    out_fr = jnp.maximum(
        jnp.dot(fr * t_fr, wfrt_ref[...], preferred_element_type=jnp.float32)
        + bfr_ref[...], 0.0)
    t_be = jnp.dot(x, wtbet_ref[...], preferred_element_type=jnp.float32) + btbe_ref[...]
    out_be = jnp.maximum(
        jnp.dot(be * t_be, wbet_ref[...], preferred_element_type=jnp.float32)
        + bbe_ref[...], 0.0)
    ball = (jnp.sum(x * wbal_ref[...], axis=1, keepdims=True)
            + bbal_ref[...][:, :1])
    bal = jax.nn.sigmoid(jnp.maximum(ball, 0.0))
    h = jnp.dot(x, wselft_ref[...], preferred_element_type=jnp.float32) + bself_ref[...]
    h = h + bal * out_be + (1.0 - bal) * out_fr
    h = h + jnp.dot(kern, wfc3t_ref[...], preferred_element_type=jnp.float32)
    out_ref[...] = jnp.where(h > 0, h, 0.01 * h)


def _post(x, p6, wtfrt, btfr, wfrt, bfr, wtbet, btbe, wbet, bbe, wbal, bbal,
          wselft, bself, wfc3t, block):
    n, d = x.shape
    grid = (n + block - 1) // block
    row = lambda i: (0, 0)
    return pl.pallas_call(
        _post_body,
        grid=(grid,),
        in_specs=[
            pl.BlockSpec((block, d), lambda i: (i, 0)),
            pl.BlockSpec((6, block, d), lambda i: (0, i, 0)),
            pl.BlockSpec((d, d), row), pl.BlockSpec((1, d), row),
            pl.BlockSpec((d, d), row), pl.BlockSpec((1, d), row),
            pl.BlockSpec((d, d), row), pl.BlockSpec((1, d), row),
            pl.BlockSpec((d, d), row), pl.BlockSpec((1, d), row),
            pl.BlockSpec((1, d), row), pl.BlockSpec((1, d), row),
            pl.BlockSpec((d, d), row), pl.BlockSpec((1, d), row),
            pl.BlockSpec((d, d), row),
        ],
        out_specs=pl.BlockSpec((block, d), lambda i: (i, 0)),
        out_shape=jax.ShapeDtypeStruct((n, d), jnp.float32),
    )(x, p6, wtfrt, btfr, wfrt, bfr, wtbet, btbe, wbet, bbe, wbal, bbal,
      wselft, bself, wfc3t)


def kernel(x, edge_index, labels, W_fc, W_fc2, W_fc3, mu, inv_sigma,
           W_fr, b_fr, Wt_fr, bt_fr, W_be, b_be, Wt_be, bt_be,
           W_bal, b_bal, W_self, b_self):
    n, d = x.shape
    e = edge_index.shape[1]

    src = edge_index[0].astype(jnp.int32)
    dst = edge_index[1].astype(jnp.int32)
    chunk = NW * EB * KU
    e_pad = ((e + chunk - 1) // chunk) * chunk
    pad = e_pad - e
    if pad:
        src = jnp.concatenate([src, jnp.zeros((pad,), jnp.int32)])
        dst = jnp.concatenate([dst, jnp.full((pad,), n, jnp.int32)])
        n_min = n + 1
    else:
        n_min = n
    n_pad = ((n_min + NS * 8 - 1) // (NS * 8)) * (NS * 8)
    nchunks = e_pad // (NW * EB)
    idxp = jnp.stack([src.reshape(-1, EB), dst.reshape(-1, EB)], axis=1)

    labm = jnp.broadcast_to((labels == 1).astype(jnp.float32)[:, None], (n, d))
    mu_row = mu.reshape(1, d)
    sg_row = inv_sigma.reshape(1, d)

    vfr, wu = _pre(x, labm, W_fc.T, W_fc2.T, mu_row, sg_row, block=1000)

    zrows = jnp.zeros((n_pad // NS, d), jnp.float32)
    seg = _make_seg(n_pad, d, e_pad, nchunks)
    p = seg(x, vfr, wu, idxp, zrows)
    p6 = p[:, :n, :] if n_pad != n else p

    rst = _post(
        x, p6,
        Wt_fr.T, bt_fr.reshape(1, d), W_fr.T, b_fr.reshape(1, d),
        Wt_be.T, bt_be.reshape(1, d), W_be.T, b_be.reshape(1, d),
        W_bal.reshape(1, d), jnp.broadcast_to(b_bal.reshape(1, 1), (1, d)),
        W_self.T, b_self.reshape(1, d), W_fc3.T, block=1000)
    return rst


# speedup vs baseline: 1.9951x; 1.0076x over previous
import functools

import jax
import jax.numpy as jnp
from jax import lax
from jax.experimental import pallas as pl
from jax.experimental.pallas import tpu as pltpu
from jax.experimental.pallas import tpu_sc as plsc

NC, NS = 2, 16
NW = NC * NS
EB = 112
KU = 3


def _pre_body(x_ref, labm_ref, wfct_ref, wfc2t_ref, mu_ref, sg_ref,
              vfr_ref, wu_ref):
    x = x_ref[...]
    z = jnp.dot(x, wfct_ref[...], preferred_element_type=jnp.float32)
    u = jnp.dot(x, wfc2t_ref[...], preferred_element_type=jnp.float32)
    t = (z - mu_ref[...]) * sg_ref[...]
    wu_ref[...] = jnp.exp(-0.5 * t * t) * u
    vfr_ref[...] = x * labm_ref[...]


def _pre(x, labm, wfct, wfc2t, mu_row, sg_row, block):
    n, d = x.shape
    grid = (n + block - 1) // block
    return pl.pallas_call(
        _pre_body,
        grid=(grid,),
        in_specs=[
            pl.BlockSpec((block, d), lambda i: (i, 0)),
            pl.BlockSpec((block, d), lambda i: (i, 0)),
            pl.BlockSpec((d, d), lambda i: (0, 0)),
            pl.BlockSpec((d, d), lambda i: (0, 0)),
            pl.BlockSpec((1, d), lambda i: (0, 0)),
            pl.BlockSpec((1, d), lambda i: (0, 0)),
        ],
        out_specs=[
            pl.BlockSpec((block, d), lambda i: (i, 0)),
            pl.BlockSpec((block, d), lambda i: (i, 0)),
        ],
        out_shape=[
            jax.ShapeDtypeStruct((n, d), jnp.float32),
            jax.ShapeDtypeStruct((n, d), jnp.float32),
        ],
    )(x, labm, wfct, wfc2t, mu_row, sg_row)


def _make_seg(n_pad, d, e_pad, nchunks):
    mesh = plsc.VectorSubcoreMesh(core_axis_name="c", subcore_axis_name="s",
                                  num_cores=NC, num_subcores=NS)
    rpw = n_pad // NS

    @functools.partial(
        pl.kernel,
        out_type=jax.ShapeDtypeStruct((3 * NC, n_pad, d), jnp.float32),
        mesh=mesh,
        scratch_types=[
            [pltpu.VMEM((2, EB), jnp.int32)] * KU,
            [pltpu.VMEM((EB, d), jnp.float32)] * KU,
            pltpu.VMEM_SHARED((n_pad, d), jnp.float32),
            [pltpu.SemaphoreType.DMA] * KU,
            [pltpu.SemaphoreType.DMA] * KU,
            [pltpu.SemaphoreType.DMA] * KU,
        ],
    )
    def seg(xt, vfrt, wut, idxp, zrows, out, ib, rows, acc, isem, gsem, ssem):
        c = lax.axis_index("c")
        s = lax.axis_index("s")
        r0 = s * rpw
        wid = c * NS + s
        cbase = wid * nchunks

        for ch, tab in enumerate((xt, vfrt, wut)):
            pltpu.sync_copy(zrows, acc.at[pl.ds(r0, rpw)])
            plsc.subcore_barrier()

            def body(k, carry, tab=tab):
                c0 = cbase + k * KU
                icpy = [pltpu.async_copy(idxp.at[c0 + j], ib[j], isem[j])
                        for j in range(KU)]
                gcpy = []
                for j in range(KU):
                    icpy[j].wait()
                    gcpy.append(pltpu.async_copy(tab.at[ib[j].at[0]], rows[j],
                                                 gsem[j]))
                scpy = []
                for j in range(KU):
                    gcpy[j].wait()
                    scpy.append(pltpu.async_copy(rows[j], acc.at[ib[j].at[1]],
                                                 ssem[j], add=True))
                for j in range(KU):
                    scpy[j].wait()
                return carry

            lax.fori_loop(0, nchunks // KU, body, 0)
            plsc.subcore_barrier()
            for cc in range(NC):
                @pl.when(c == cc)
                def _(ch=ch, cc=cc):
                    pltpu.sync_copy(acc.at[pl.ds(r0, rpw)],
                                    out.at[ch * NC + cc, pl.ds(r0, rpw)])
            plsc.subcore_barrier()

    return seg


def _post_body(x_ref, p_ref, wtfrt_ref, btfr_ref, wfrt_ref, bfr_ref,
               wtbet_ref, btbe_ref, wbet_ref, bbe_ref, wbal_ref, bbal_ref,
               wselft_ref, bself_ref, wfc3t_ref, out_ref):
    x = x_ref[...]
    tot = p_ref[0] + p_ref[1]
    fr = p_ref[2] + p_ref[3]
    kern = p_ref[4] + p_ref[5]
    be = tot - fr
    t_fr = jnp.dot(x, wtfrt_ref[...], preferred_element_type=jnp.float32) + btfr_ref[...]
    out_fr = jnp.maximum(
        jnp.dot(fr * t_fr, wfrt_ref[...], preferred_element_type=jnp.float32)
        + bfr_ref[...], 0.0)
    t_be = jnp.dot(x, wtbet_ref[...], preferred_element_type=jnp.float32) + btbe_ref[...]
    out_be = jnp.maximum(
        jnp.dot(be * t_be, wbet_ref[...], preferred_element_type=jnp.float32)
        + bbe_ref[...], 0.0)
    ball = (jnp.sum(x * wbal_ref[...], axis=1, keepdims=True)
            + bbal_ref[...][:, :1])
    bal = jax.nn.sigmoid(jnp.maximum(ball, 0.0))
    h = jnp.dot(x, wselft_ref[...], preferred_element_type=jnp.float32) + bself_ref[...]
    h = h + bal * out_be + (1.0 - bal) * out_fr
    h = h + jnp.dot(kern, wfc3t_ref[...], preferred_element_type=jnp.float32)
    out_ref[...] = jnp.where(h > 0, h, 0.01 * h)


def _post(x, p6, wtfrt, btfr, wfrt, bfr, wtbet, btbe, wbet, bbe, wbal, bbal,
          wselft, bself, wfc3t, block):
    n, d = x.shape
    grid = (n + block - 1) // block
    row = lambda i: (0, 0)
    return pl.pallas_call(
        _post_body,
        grid=(grid,),
        in_specs=[
            pl.BlockSpec((block, d), lambda i: (i, 0)),
            pl.BlockSpec((6, block, d), lambda i: (0, i, 0)),
            pl.BlockSpec((d, d), row), pl.BlockSpec((1, d), row),
            pl.BlockSpec((d, d), row), pl.BlockSpec((1, d), row),
            pl.BlockSpec((d, d), row), pl.BlockSpec((1, d), row),
            pl.BlockSpec((d, d), row), pl.BlockSpec((1, d), row),
            pl.BlockSpec((1, d), row), pl.BlockSpec((1, d), row),
            pl.BlockSpec((d, d), row), pl.BlockSpec((1, d), row),
            pl.BlockSpec((d, d), row),
        ],
        out_specs=pl.BlockSpec((block, d), lambda i: (i, 0)),
        out_shape=jax.ShapeDtypeStruct((n, d), jnp.float32),
    )(x, p6, wtfrt, btfr, wfrt, bfr, wtbet, btbe, wbet, bbe, wbal, bbal,
      wselft, bself, wfc3t)


def kernel(x, edge_index, labels, W_fc, W_fc2, W_fc3, mu, inv_sigma,
           W_fr, b_fr, Wt_fr, bt_fr, W_be, b_be, Wt_be, bt_be,
           W_bal, b_bal, W_self, b_self):
    n, d = x.shape
    e = edge_index.shape[1]

    src = edge_index[0].astype(jnp.int32)
    dst = edge_index[1].astype(jnp.int32)
    chunk = NW * EB * KU
    e_pad = ((e + chunk - 1) // chunk) * chunk
    pad = e_pad - e
    if pad:
        src = jnp.concatenate([src, jnp.zeros((pad,), jnp.int32)])
        dst = jnp.concatenate([dst, jnp.full((pad,), n, jnp.int32)])
        n_min = n + 1
    else:
        n_min = n
    n_pad = ((n_min + NS * 8 - 1) // (NS * 8)) * (NS * 8)
    nchunks = e_pad // (NW * EB)
    idxp = jnp.stack([src.reshape(-1, EB), dst.reshape(-1, EB)], axis=1)

    labm = jnp.broadcast_to((labels == 1).astype(jnp.float32)[:, None], (n, d))
    mu_row = mu.reshape(1, d)
    sg_row = inv_sigma.reshape(1, d)

    vfr, wu = _pre(x, labm, W_fc.T, W_fc2.T, mu_row, sg_row, block=1000)

    zrows = jnp.zeros((n_pad // NS, d), jnp.float32)
    seg = _make_seg(n_pad, d, e_pad, nchunks)
    p = seg(x, vfr, wu, idxp, zrows)
    p6 = p[:, :n, :] if n_pad != n else p

    rst = _post(
        x, p6,
        Wt_fr.T, bt_fr.reshape(1, d), W_fr.T, b_fr.reshape(1, d),
        Wt_be.T, bt_be.reshape(1, d), W_be.T, b_be.reshape(1, d),
        W_bal.reshape(1, d), jnp.broadcast_to(b_bal.reshape(1, 1), (1, d)),
        W_self.T, b_self.reshape(1, d), W_fc3.T, block=1000)
    return rst
